# baseline (speedup 1.0000x reference)
import functools

import numpy as np
import jax
import jax.numpy as jnp
from jax import lax
from jax.experimental import pallas as pl
from jax.experimental.pallas import tpu as pltpu
from jax.experimental.pallas import tpu_sc as plsc
from jax.experimental.shard_map import shard_map
from jax.sharding import Mesh, PartitionSpec as P

POP = 100000
NSAMP = 2048
R = 8
C = 4096
NCB = 25
PP = NCB * C
NSB = NSAMP // R

_NDEV = 2 if len(jax.devices()) >= 2 else 1
_NSB_L = NSB // _NDEV
_NSAMP_L = NSAMP // _NDEV

_TINY = np.float32(1.1754943508222875e-38)
_ONE_BITS = np.uint32(0x3F800000)
_KS0 = np.uint32(0)
_KS1 = np.uint32(42)
_KS2 = np.uint32(0x1BD11BDA ^ 42)
_INJ = ((_KS1, np.uint32(_KS2 + np.uint32(1))),
        (_KS2, np.uint32(_KS0 + np.uint32(2))),
        (_KS0, np.uint32(_KS1 + np.uint32(3))),
        (_KS1, np.uint32(_KS2 + np.uint32(4))),
        (_KS2, np.uint32(_KS0 + np.uint32(5))))
_ROT = ((13, 15, 26, 6), (17, 29, 16, 24))


def _rotl(x, r):
    return (x << r) | (x >> (32 - r))


def _threefry_bits(q):
    x1 = q
    x0 = x1
    x1 = x0 ^ _rotl(x1, 13)
    for r in _ROT[0][1:]:
        x0 = x0 + x1
        x1 = x0 ^ _rotl(x1, r)
    x0 = x0 + _INJ[0][0]
    x1 = x1 + _INJ[0][1]
    for g in range(1, 5):
        a, b = _INJ[g]
        for r in _ROT[g % 2]:
            x0 = x0 + x1
            x1 = x0 ^ _rotl(x1, r)
        x0 = x0 + a
        x1 = x1 + b
    return x0 ^ x1


def _sampler_body(base_ref, a_ref, out_ref):
    sb = pl.program_id(0) + base_ref[0]

    row = lax.broadcasted_iota(jnp.uint32, (R, 128), 0)
    lane_u = lax.broadcasted_iota(jnp.uint32, (R, 128), 1)
    lane_i = lax.broadcasted_iota(jnp.int32, (R, 128), 1)
    s0 = np.uint32(R) * sb.astype(jnp.uint32)
    qrl = (s0 + row) * np.uint32(POP) + lane_u + _KS1

    def step(cb, carry):
        s_run, i_run = carry
        base = cb * C
        ss = []
        ii = []
        for k in range(C // 128):
            off = base + k * 128
            q = qrl + off.astype(jnp.uint32)
            bits = _threefry_bits(q)
            f = lax.bitcast_convert_type(
                (bits >> 9) | _ONE_BITS, jnp.float32) - 1.0
            u = jnp.maximum(f, _TINY)
            score = (-jnp.log(u)) * jnp.exp(a_ref[cb, :, k * 128:(k + 1) * 128])
            ss.append(score)
            ii.append(off + lane_i)

        while len(ss) > 1:
            ns, ni = [], []
            for j in range(0, len(ss) - 1, 2):
                take = ss[j + 1] < ss[j]
                ni.append(jnp.where(take, ii[j + 1], ii[j]))
                ns.append(jnp.where(take, ss[j + 1], ss[j]))
            if len(ss) % 2:
                ns.append(ss[-1])
                ni.append(ii[-1])
            ss, ii = ns, ni
        take = ss[0] < s_run
        i_run = jnp.where(take, ii[0], i_run)
        s_run = jnp.where(take, ss[0], s_run)
        return (s_run, i_run)

    s_init = jnp.full((R, 128), jnp.inf, jnp.float32)
    i_init = jnp.zeros((R, 128), jnp.int32)
    s_fin, i_fin = lax.fori_loop(0, NCB, step, (s_init, i_init))

    m = jnp.min(s_fin, axis=1, keepdims=True)
    masked = jnp.where(s_fin == m, i_fin, jnp.int32(0x7FFFFFFF))
    idx = jnp.min(masked, axis=1, keepdims=True)
    out_ref[0] = jnp.broadcast_to(idx, (R, 128))


def _build_sampler(interpret=False):
    return pl.pallas_call(
        _sampler_body,
        grid=(_NSB_L,),
        in_specs=[pl.BlockSpec(memory_space=pltpu.SMEM),
                  pl.BlockSpec((NCB, R, C), lambda sb: (0, 0, 0))],
        out_specs=pl.BlockSpec((1, R, 128), lambda sb: (sb, 0, 0)),
        out_shape=jax.ShapeDtypeStruct((_NSB_L, R, 128), jnp.int32),
        compiler_params=pltpu.CompilerParams(
            dimension_semantics=("arbitrary",)),
        interpret=interpret,
    )


_SAMPLER = _build_sampler()


def _sc_gather(table, idx, n):
    info = plsc.get_sparse_core_info()
    nw = info.num_cores * info.num_subcores
    per = n // nw
    mesh = plsc.VectorSubcoreMesh(core_axis_name="c", subcore_axis_name="s")

    @functools.partial(
        pl.kernel, mesh=mesh,
        out_type=jax.ShapeDtypeStruct((n,), jnp.float32),
        scratch_types=[
            pltpu.VMEM((per,), jnp.int32),
            pltpu.VMEM((per,), jnp.float32),
            pltpu.SemaphoreType.DMA,
        ],
    )
    def gather_kernel(table_hbm, idx_hbm, out_hbm, idx_v, vals_v, sem):
        wid = lax.axis_index("s") * info.num_cores + lax.axis_index("c")
        base = wid * per
        pltpu.sync_copy(idx_hbm.at[pl.ds(base, per)], idx_v)
        pltpu.async_copy(table_hbm.at[idx_v], vals_v, sem).wait()
        pltpu.sync_copy(vals_v, out_hbm.at[pl.ds(base, per)])

    return gather_kernel(table, idx)


def _sample_and_gather(assessment, a_bc):
    if _NDEV == 1:
        base = jnp.zeros((1,), jnp.int32)
    else:
        base = jnp.reshape(
            lax.axis_index("d").astype(jnp.int32) * _NSB_L, (1,))
    sel_blk = _SAMPLER(base, a_bc)
    sel = sel_blk[:, :, 0].reshape(_NSAMP_L)
    vals = _sc_gather(assessment, sel, _NSAMP_L)
    return sel, vals


def kernel(assessment):
    a_pad = jnp.concatenate(
        [assessment, jnp.full((PP - POP,), jnp.inf, jnp.float32)])
    a_bc = jnp.broadcast_to(a_pad.reshape(NCB, 1, C), (NCB, R, C))
    if _NDEV == 1:
        sel, vals = _sample_and_gather(assessment, a_bc)
    else:
        mesh = Mesh(np.array(jax.devices()[:_NDEV]), ("d",))
        sel, vals = shard_map(
            _sample_and_gather, mesh=mesh,
            in_specs=(P(), P()), out_specs=P("d"),
            check_rep=False,
        )(assessment, a_bc)
    selm = sel.reshape(1024, 2).astype(jnp.int64)
    valm = vals.reshape(1024, 2)
    return (valm[:, 0], selm[:, 0], valm[:, 1], selm[:, 1])


# speedup vs baseline: 1.0147x; 1.0147x over previous
import functools

import numpy as np
import jax
import jax.numpy as jnp
from jax import lax
from jax.experimental import pallas as pl
from jax.experimental.pallas import tpu as pltpu
from jax.experimental.pallas import tpu_sc as plsc
from jax.experimental.shard_map import shard_map
from jax.sharding import Mesh, PartitionSpec as P

POP = 100000
NSAMP = 2048
R = 8
C = 3584
NCB = 28
PP = NCB * C
NSB = NSAMP // R

_NDEV = 2 if len(jax.devices()) >= 2 else 1
_NSB_L = NSB // _NDEV
_NSAMP_L = NSAMP // _NDEV

_TINY = np.float32(1.1754943508222875e-38)
_ONE_BITS = np.uint32(0x3F800000)
_KS0 = np.uint32(0)
_KS1 = np.uint32(42)
_KS2 = np.uint32(0x1BD11BDA ^ 42)
_INJ = ((_KS1, np.uint32(_KS2 + np.uint32(1))),
        (_KS2, np.uint32(_KS0 + np.uint32(2))),
        (_KS0, np.uint32(_KS1 + np.uint32(3))),
        (_KS1, np.uint32(_KS2 + np.uint32(4))),
        (_KS2, np.uint32(_KS0 + np.uint32(5))))
_ROT = ((13, 15, 26, 6), (17, 29, 16, 24))


def _rotl(x, r):
    return (x << r) | (x >> (32 - r))


def _threefry_bits(q):
    x1 = q
    x0 = x1
    x1 = x0 ^ _rotl(x1, 13)
    for r in _ROT[0][1:]:
        x0 = x0 + x1
        x1 = x0 ^ _rotl(x1, r)
    x0 = x0 + _INJ[0][0]
    x1 = x1 + _INJ[0][1]
    for g in range(1, 5):
        a, b = _INJ[g]
        for r in _ROT[g % 2]:
            x0 = x0 + x1
            x1 = x0 ^ _rotl(x1, r)
        x0 = x0 + a
        x1 = x1 + b
    return x0 ^ x1


def _sampler_body(base_ref, b_ref, out_ref):
    sb = pl.program_id(0) + base_ref[0]

    row = lax.broadcasted_iota(jnp.uint32, (R, 128), 0)
    lane_u = lax.broadcasted_iota(jnp.uint32, (R, 128), 1)
    lane_i = lax.broadcasted_iota(jnp.int32, (R, 128), 1)
    s0 = np.uint32(R) * sb.astype(jnp.uint32)
    qrl = (s0 + row) * np.uint32(POP) + lane_u + _KS1

    def step(cb, carry):
        s_run, i_run = carry
        base = cb * C
        ss = []
        ii = []
        for k in range(C // 128):
            off = base + k * 128
            q = qrl + off.astype(jnp.uint32)
            bits = _threefry_bits(q)
            f = lax.bitcast_convert_type(
                (bits >> 9) | _ONE_BITS, jnp.float32) - 1.0
            u = jnp.maximum(f, _TINY)
            score = jnp.log2(u) * jnp.exp2(b_ref[cb, :, k * 128:(k + 1) * 128])
            ss.append(score)
            ii.append(off + lane_i)

        while len(ss) > 1:
            ns, ni = [], []
            for j in range(0, len(ss) - 1, 2):
                take = ss[j + 1] > ss[j]
                ni.append(jnp.where(take, ii[j + 1], ii[j]))
                ns.append(jnp.where(take, ss[j + 1], ss[j]))
            if len(ss) % 2:
                ns.append(ss[-1])
                ni.append(ii[-1])
            ss, ii = ns, ni
        take = ss[0] > s_run
        i_run = jnp.where(take, ii[0], i_run)
        s_run = jnp.where(take, ss[0], s_run)
        return (s_run, i_run)

    s_init = jnp.full((R, 128), -jnp.inf, jnp.float32)
    i_init = jnp.zeros((R, 128), jnp.int32)
    s_fin, i_fin = lax.fori_loop(0, NCB, step, (s_init, i_init))

    m = jnp.max(s_fin, axis=1, keepdims=True)
    masked = jnp.where(s_fin == m, i_fin, jnp.int32(0x7FFFFFFF))
    idx = jnp.min(masked, axis=1, keepdims=True)
    out_ref[0] = jnp.broadcast_to(idx, (R, 128))


def _build_sampler(interpret=False):
    return pl.pallas_call(
        _sampler_body,
        grid=(_NSB_L,),
        in_specs=[pl.BlockSpec(memory_space=pltpu.SMEM),
                  pl.BlockSpec((NCB, R, C), lambda sb: (0, 0, 0))],
        out_specs=pl.BlockSpec((1, R, 128), lambda sb: (sb, 0, 0)),
        out_shape=jax.ShapeDtypeStruct((_NSB_L, R, 128), jnp.int32),
        compiler_params=pltpu.CompilerParams(
            dimension_semantics=("arbitrary",)),
        interpret=interpret,
    )


_SAMPLER = _build_sampler()


def _sc_gather(table, idx, n):
    info = plsc.get_sparse_core_info()
    nw = info.num_cores * info.num_subcores
    per = n // nw
    mesh = plsc.VectorSubcoreMesh(core_axis_name="c", subcore_axis_name="s")

    @functools.partial(
        pl.kernel, mesh=mesh,
        out_type=jax.ShapeDtypeStruct((n,), jnp.float32),
        scratch_types=[
            pltpu.VMEM((per,), jnp.int32),
            pltpu.VMEM((per,), jnp.float32),
            pltpu.SemaphoreType.DMA,
        ],
    )
    def gather_kernel(table_hbm, idx_hbm, out_hbm, idx_v, vals_v, sem):
        wid = lax.axis_index("s") * info.num_cores + lax.axis_index("c")
        base = wid * per
        pltpu.sync_copy(idx_hbm.at[pl.ds(base, per)], idx_v)
        pltpu.async_copy(table_hbm.at[idx_v], vals_v, sem).wait()
        pltpu.sync_copy(vals_v, out_hbm.at[pl.ds(base, per)])

    return gather_kernel(table, idx)


def _sample_and_gather(assessment, b_bc):
    if _NDEV == 1:
        base = jnp.zeros((1,), jnp.int32)
    else:
        base = jnp.reshape(
            lax.axis_index("d").astype(jnp.int32) * _NSB_L, (1,))
    sel_blk = _SAMPLER(base, b_bc)
    sel = sel_blk[:, :, 0].reshape(_NSAMP_L)
    vals = _sc_gather(assessment, sel, _NSAMP_L)
    return sel, vals


def kernel(assessment):
    b_pad = jnp.concatenate(
        [assessment * np.float32(1.4426950408889634),
         jnp.full((PP - POP,), jnp.inf, jnp.float32)])
    b_bc = jnp.broadcast_to(b_pad.reshape(NCB, 1, C), (NCB, R, C))
    if _NDEV == 1:
        sel, vals = _sample_and_gather(assessment, b_bc)
    else:
        mesh = Mesh(np.array(jax.devices()[:_NDEV]), ("d",))
        sel, vals = shard_map(
            _sample_and_gather, mesh=mesh,
            in_specs=(P(), P()), out_specs=P("d"),
            check_rep=False,
        )(assessment, b_bc)
    selm = sel.reshape(1024, 2).astype(jnp.int64)
    valm = vals.reshape(1024, 2)
    return (valm[:, 0], selm[:, 0], valm[:, 1], selm[:, 1])


# speedup vs baseline: 1.0183x; 1.0035x over previous
import functools

import numpy as np
import jax
import jax.numpy as jnp
from jax import lax
from jax.experimental import pallas as pl
from jax.experimental.pallas import tpu as pltpu
from jax.experimental.pallas import tpu_sc as plsc
from jax.experimental.shard_map import shard_map
from jax.sharding import Mesh, PartitionSpec as P

POP = 100000
NSAMP = 2048
R = 8
C = 3584
NCB = 28
PP = NCB * C
NSB = NSAMP // R

_NDEV = 2 if len(jax.devices()) >= 2 else 1
_NSB_L = NSB // _NDEV
_NSAMP_L = NSAMP // _NDEV

_TINY = np.float32(1.1754943508222875e-38)
_ONE_BITS = np.uint32(0x3F800000)
_KS0 = np.uint32(0)
_KS1 = np.uint32(42)
_KS2 = np.uint32(0x1BD11BDA ^ 42)
_INJ = ((_KS1, np.uint32(_KS2 + np.uint32(1))),
        (_KS2, np.uint32(_KS0 + np.uint32(2))),
        (_KS0, np.uint32(_KS1 + np.uint32(3))),
        (_KS1, np.uint32(_KS2 + np.uint32(4))),
        (_KS2, np.uint32(_KS0 + np.uint32(5))))
_ROT = ((13, 15, 26, 6), (17, 29, 16, 24))


def _rotl(x, r):
    return (x << r) | (x >> (32 - r))


def _threefry_bits(q):
    x1 = q
    x0 = x1
    x1 = x0 ^ _rotl(x1, 13)
    for r in _ROT[0][1:]:
        x0 = x0 + x1
        x1 = x0 ^ _rotl(x1, r)
    x0 = x0 + _INJ[0][0]
    x1 = x1 + _INJ[0][1]
    for g in range(1, 5):
        a, b = _INJ[g]
        for r in _ROT[g % 2]:
            x0 = x0 + x1
            x1 = x0 ^ _rotl(x1, r)
        x0 = x0 + a
        x1 = x1 + b
    return x0 ^ x1


def _sampler_body(base_ref, b_ref, out_ref):
    sb = pl.program_id(0) + base_ref[0]

    row = lax.broadcasted_iota(jnp.uint32, (R, 128), 0)
    lane_u = lax.broadcasted_iota(jnp.uint32, (R, 128), 1)
    lane_i = lax.broadcasted_iota(jnp.int32, (R, 128), 1)
    s0 = np.uint32(R) * sb.astype(jnp.uint32)
    qrl = (s0 + row) * np.uint32(POP) + lane_u + _KS1

    def step(cb, carry):
        s_run, i_run = carry
        base = cb * C
        ss = []
        ii = []
        for k in range(C // 128):
            off = base + k * 128
            q = qrl + off.astype(jnp.uint32)
            bits = _threefry_bits(q)
            f = lax.bitcast_convert_type(
                (bits >> 9) | _ONE_BITS, jnp.float32) - 1.0
            u = jnp.maximum(f, _TINY)
            score = jnp.log2(u) * jnp.exp2(b_ref[cb, :, k * 128:(k + 1) * 128])
            ss.append(score)
            ii.append(off + lane_i)

        while len(ss) > 1:
            ns, ni = [], []
            for j in range(0, len(ss) - 1, 2):
                take = ss[j + 1] > ss[j]
                ni.append(jnp.where(take, ii[j + 1], ii[j]))
                ns.append(jnp.where(take, ss[j + 1], ss[j]))
            if len(ss) % 2:
                ns.append(ss[-1])
                ni.append(ii[-1])
            ss, ii = ns, ni
        take = ss[0] > s_run
        i_run = jnp.where(take, ii[0], i_run)
        s_run = jnp.where(take, ss[0], s_run)
        return (s_run, i_run)

    def step2(h, carry):
        return step(2 * h + 1, step(2 * h, carry))

    s_init = jnp.full((R, 128), -jnp.inf, jnp.float32)
    i_init = jnp.zeros((R, 128), jnp.int32)
    s_fin, i_fin = lax.fori_loop(0, NCB // 2, step2, (s_init, i_init))

    m = jnp.max(s_fin, axis=1, keepdims=True)
    masked = jnp.where(s_fin == m, i_fin, jnp.int32(0x7FFFFFFF))
    idx = jnp.min(masked, axis=1, keepdims=True)
    out_ref[0] = jnp.broadcast_to(idx, (R, 128))


def _build_sampler(interpret=False):
    return pl.pallas_call(
        _sampler_body,
        grid=(_NSB_L,),
        in_specs=[pl.BlockSpec(memory_space=pltpu.SMEM),
                  pl.BlockSpec((NCB, R, C), lambda sb: (0, 0, 0))],
        out_specs=pl.BlockSpec((1, R, 128), lambda sb: (sb, 0, 0)),
        out_shape=jax.ShapeDtypeStruct((_NSB_L, R, 128), jnp.int32),
        compiler_params=pltpu.CompilerParams(
            dimension_semantics=("arbitrary",)),
        interpret=interpret,
    )


_SAMPLER = _build_sampler()


def _sc_gather(table, idx, n):
    info = plsc.get_sparse_core_info()
    nw = info.num_cores * info.num_subcores
    per = n // nw
    mesh = plsc.VectorSubcoreMesh(core_axis_name="c", subcore_axis_name="s")

    @functools.partial(
        pl.kernel, mesh=mesh,
        out_type=jax.ShapeDtypeStruct((n,), jnp.float32),
        scratch_types=[
            pltpu.VMEM((per,), jnp.int32),
            pltpu.VMEM((per,), jnp.float32),
            pltpu.SemaphoreType.DMA,
        ],
    )
    def gather_kernel(table_hbm, idx_hbm, out_hbm, idx_v, vals_v, sem):
        wid = lax.axis_index("s") * info.num_cores + lax.axis_index("c")
        base = wid * per
        pltpu.sync_copy(idx_hbm.at[pl.ds(base, per)], idx_v)
        pltpu.async_copy(table_hbm.at[idx_v], vals_v, sem).wait()
        pltpu.sync_copy(vals_v, out_hbm.at[pl.ds(base, per)])

    return gather_kernel(table, idx)


def _sample_and_gather(assessment, b_bc):
    if _NDEV == 1:
        base = jnp.zeros((1,), jnp.int32)
    else:
        base = jnp.reshape(
            lax.axis_index("d").astype(jnp.int32) * _NSB_L, (1,))
    sel_blk = _SAMPLER(base, b_bc)
    sel = sel_blk[:, :, 0].reshape(_NSAMP_L)
    vals = _sc_gather(assessment, sel, _NSAMP_L)
    return sel, vals


def kernel(assessment):
    b_pad = jnp.concatenate(
        [assessment * np.float32(1.4426950408889634),
         jnp.full((PP - POP,), jnp.inf, jnp.float32)])
    b_bc = jnp.broadcast_to(b_pad.reshape(NCB, 1, C), (NCB, R, C))
    if _NDEV == 1:
        sel, vals = _sample_and_gather(assessment, b_bc)
    else:
        mesh = Mesh(np.array(jax.devices()[:_NDEV]), ("d",))
        sel, vals = shard_map(
            _sample_and_gather, mesh=mesh,
            in_specs=(P(), P()), out_specs=P("d"),
            check_rep=False,
        )(assessment, b_bc)
    selm = sel.reshape(1024, 2).astype(jnp.int64)
    valm = vals.reshape(1024, 2)
    return (valm[:, 0], selm[:, 0], valm[:, 1], selm[:, 1])
